# baseline (speedup 1.0000x reference)
import functools

import jax
import jax.numpy as jnp
from jax import lax
from jax.experimental import pallas as pl
from jax.experimental.pallas import tpu as pltpu
from jax.experimental.pallas import tpu_sc as plsc

_H = 256
_W = 256
_F = 32


_NC = 2
_NS = 16
_ROW = _H * _W // 2
_PER_W = 96 // (_NC * _NS)


def _sc_gather_body(frames_hbm, out_hbm, buf0, buf1, buf2, sem0, sem1, sem2):
    wid = lax.axis_index("s") * _NC + lax.axis_index("c")
    bufs = (buf0, buf1, buf2)
    sems = (sem0, sem1, sem2)

    def src_row(i):
        h = wid * _PER_W + i
        q = lax.div(h, 2)
        half = h - 2 * q
        c = lax.div(q, 16)
        j = q - 16 * c
        return 2 * (c * 64 + lax.div(63 * j, 15)) + half

    reads = [
        pltpu.async_copy(
            frames_hbm.at[pl.ds(src_row(i), 1)], bufs[i], sems[i]
        )
        for i in range(_PER_W)
    ]
    for i in range(_PER_W):
        reads[i].wait()
        h = wid * _PER_W + i
        pltpu.sync_copy(bufs[i], out_hbm.at[pl.ds(h, 1)])


_sc_gather = functools.partial(
    pl.kernel,
    mesh=plsc.VectorSubcoreMesh(
        core_axis_name="c", subcore_axis_name="s",
        num_cores=_NC, num_subcores=_NS,
    ),
    out_type=jax.ShapeDtypeStruct((96, _ROW), jnp.float32),
    scratch_types=[
        pltpu.VMEM((1, _ROW), jnp.float32),
        pltpu.VMEM((1, _ROW), jnp.float32),
        pltpu.VMEM((1, _ROW), jnp.float32),
        pltpu.SemaphoreType.DMA,
        pltpu.SemaphoreType.DMA,
        pltpu.SemaphoreType.DMA,
    ],
)(_sc_gather_body)


def _copy_body(in_ref, fast_ref):
    fast_ref[...] = in_ref[...]


def _tc_copy(frames_flat):
    n_blocks = frames_flat.shape[0] // _F
    return pl.pallas_call(
        _copy_body,
        grid=(n_blocks,),
        in_specs=[pl.BlockSpec((_F, _H, _W), lambda k: (k, 0, 0))],
        out_specs=pl.BlockSpec((_F, _H, _W), lambda k: (k, 0, 0)),
        out_shape=jax.ShapeDtypeStruct((n_blocks * _F, _H, _W), jnp.float32),
        compiler_params=pltpu.CompilerParams(
            dimension_semantics=("arbitrary",),
        ),
    )(frames_flat)


def kernel(frames):
    c, t, h, w = frames.shape
    slow = _sc_gather(frames.reshape(c * t * 2, _ROW))
    fast = _tc_copy(frames.reshape(c * t, h, w))
    return (
        slow.reshape(c, t // 4, h, w),
        fast.reshape(c, t, h, w),
    )


# speedup vs baseline: 2.2492x; 2.2492x over previous
import functools

import jax
import jax.numpy as jnp
from jax import lax
from jax.experimental import pallas as pl
from jax.experimental.pallas import tpu as pltpu
from jax.experimental.pallas import tpu_sc as plsc

_H = 256
_W = 256
_F = 32


_NC = 2
_NS = 16
_PER_W = 96 // (_NC * _NS)


def _sc_gather_body(frames_hbm, out_hbm, buf0, buf1, buf2, sem0, sem1, sem2):
    wid = lax.axis_index("s") * _NC + lax.axis_index("c")
    bufs = (buf0, buf1, buf2)
    sems = (sem0, sem1, sem2)

    def coords(i):
        h = wid * _PER_W + i
        q = lax.div(h, 2)
        half = h - 2 * q
        c = lax.div(q, 16)
        j = q - 16 * c
        src = c * 64 + lax.div(63 * j, 15)
        return src, q, half * (_H // 2)

    reads = []
    for i in range(_PER_W):
        src, _, r0 = coords(i)
        reads.append(
            pltpu.async_copy(
                frames_hbm.at[pl.ds(src, 1), pl.ds(r0, _H // 2)],
                bufs[i],
                sems[i],
            )
        )
    for i in range(_PER_W):
        reads[i].wait()
        _, q, r0 = coords(i)
        pltpu.sync_copy(
            bufs[i], out_hbm.at[pl.ds(q, 1), pl.ds(r0, _H // 2)]
        )


_sc_gather = functools.partial(
    pl.kernel,
    mesh=plsc.VectorSubcoreMesh(
        core_axis_name="c", subcore_axis_name="s",
        num_cores=_NC, num_subcores=_NS,
    ),
    out_type=jax.ShapeDtypeStruct((48, _H, _W), jnp.float32),
    scratch_types=[
        pltpu.VMEM((1, _H // 2, _W), jnp.float32),
        pltpu.VMEM((1, _H // 2, _W), jnp.float32),
        pltpu.VMEM((1, _H // 2, _W), jnp.float32),
        pltpu.SemaphoreType.DMA,
        pltpu.SemaphoreType.DMA,
        pltpu.SemaphoreType.DMA,
    ],
)(_sc_gather_body)


def _copy_body(in_ref, fast_ref):
    fast_ref[...] = in_ref[...]


def _tc_copy(frames_flat):
    n_blocks = frames_flat.shape[0] // _F
    return pl.pallas_call(
        _copy_body,
        grid=(n_blocks,),
        in_specs=[pl.BlockSpec((_F, _H, _W), lambda k: (k, 0, 0))],
        out_specs=pl.BlockSpec((_F, _H, _W), lambda k: (k, 0, 0)),
        out_shape=jax.ShapeDtypeStruct((n_blocks * _F, _H, _W), jnp.float32),
        compiler_params=pltpu.CompilerParams(
            dimension_semantics=("arbitrary",),
        ),
    )(frames_flat)


def kernel(frames):
    c, t, h, w = frames.shape
    flat = frames.reshape(c * t, h, w)
    slow = _sc_gather(flat)
    fast = _tc_copy(flat)
    return (
        slow.reshape(c, t // 4, h, w),
        fast.reshape(c, t, h, w),
    )


# speedup vs baseline: 3.5463x; 1.5766x over previous
import jax
import jax.numpy as jnp
from jax.experimental import pallas as pl
from jax.experimental.pallas import tpu as pltpu

_H = 256
_W = 256
_F = 32


def _pack_body(in_ref, slow_ref, fast_ref):
    fast_ref[...] = in_ref[...]
    k = pl.program_id(0)
    for s in range(_F // 4):
        jg = k * (_F // 4) + s
        j = jax.lax.rem(jg, 16)
        c = jax.lax.div(jg, 16)
        off = c * 64 + jax.lax.div(63 * j, 15) - k * _F
        slow_ref[pl.ds(s, 1)] = in_ref[pl.ds(off, 1)]


def _pack(frames_flat):
    n_blocks = frames_flat.shape[0] // _F
    return pl.pallas_call(
        _pack_body,
        grid=(n_blocks,),
        in_specs=[pl.BlockSpec((_F, _H, _W), lambda k: (k, 0, 0))],
        out_specs=[
            pl.BlockSpec((_F // 4, _H, _W), lambda k: (k, 0, 0)),
            pl.BlockSpec((_F, _H, _W), lambda k: (k, 0, 0)),
        ],
        out_shape=[
            jax.ShapeDtypeStruct((n_blocks * (_F // 4), _H, _W), jnp.float32),
            jax.ShapeDtypeStruct((n_blocks * _F, _H, _W), jnp.float32),
        ],
        compiler_params=pltpu.CompilerParams(
            dimension_semantics=("arbitrary",),
        ),
    )(frames_flat)


def kernel(frames):
    c, t, h, w = frames.shape
    flat = frames.reshape(c * t, h, w)
    slow, fast = _pack(flat)
    return (
        slow.reshape(c, t // 4, h, w),
        fast.reshape(c, t, h, w),
    )
